# baseline (speedup 1.0000x reference)
import functools

import jax
import jax.numpy as jnp
from jax import lax
from jax.experimental import pallas as pl
from jax.experimental.pallas import tpu as pltpu
from jax.experimental.pallas import tpu_sc as plsc

_B = 4
_N = 2048
_NQ = _B * _N
_L = 16


def _ladder(m1, m2, m3, d):
    t = jnp.maximum(m1, d)
    m1 = jnp.minimum(m1, d)
    t2 = jnp.maximum(m2, t)
    m2 = jnp.minimum(m2, t)
    m3 = jnp.minimum(m3, t2)
    return m1, m2, m3


def _sc_knn(p1t, p2t, c1t, out, kx, ky, kz, cr, cg, cb, qx, qy, qz,
            db0, db1, db2, db3, a2r, a2g, a2b, a3r, a3g, a3b,
            ovr, ovg, ovb):
    nw = 32
    qpw = _NQ // nw
    wid = lax.axis_index("c") * 16 + lax.axis_index("s")
    b = wid // (_N // qpw)
    qoff = (wid % (_N // qpw)) * qpw

    pltpu.sync_copy(p1t.at[pl.ds((b * 3 + 0) * _N, _N)], kx)
    pltpu.sync_copy(p1t.at[pl.ds((b * 3 + 1) * _N, _N)], ky)
    pltpu.sync_copy(p1t.at[pl.ds((b * 3 + 2) * _N, _N)], kz)
    pltpu.sync_copy(c1t.at[pl.ds((b * 3 + 0) * _N, _N)], cr)
    pltpu.sync_copy(c1t.at[pl.ds((b * 3 + 1) * _N, _N)], cg)
    pltpu.sync_copy(c1t.at[pl.ds((b * 3 + 2) * _N, _N)], cb)
    pltpu.sync_copy(p2t.at[pl.ds((b * 3 + 0) * _N + qoff, qpw)], qx)
    pltpu.sync_copy(p2t.at[pl.ds((b * 3 + 1) * _N + qoff, qpw)], qy)
    pltpu.sync_copy(p2t.at[pl.ds((b * 3 + 2) * _N + qoff, qpw)], qz)

    inf16 = jnp.full((_L,), jnp.inf, jnp.float32)
    zero16 = jnp.zeros((_L,), jnp.float32)
    one16 = jnp.ones((_L,), jnp.float32)
    izero = jnp.zeros((_L,), jnp.int32)

    lane = lax.iota(jnp.int32, _L)
    ninf16 = jnp.full((_L,), -jnp.inf, jnp.float32)

    _dn = lax.GatherDimensionNumbers(
        offset_dims=(), collapsed_slice_dims=(0,), start_index_map=(0,))

    def _shuf(v, idx):
        return lax.gather(v, idx[:, None], _dn, (1,),
                          mode=lax.GatherScatterMode.PROMISE_IN_BOUNDS)

    def _bfly(v, op):
        for s in (8, 4, 2, 1):
            v = op(v, _shuf(v, lane ^ s))
        return v

    def _qcoord(qv, g, sel):
        return _bfly(jnp.where(sel, qv[pl.ds(g * _L, _L)], ninf16),
                     jnp.maximum)

    def _thresh(m1, m2, m3):
        r1 = _bfly(m1, jnp.minimum)
        e1 = m1 == r1
        m1 = jnp.where(e1, m2, m1)
        m2 = jnp.where(e1, m3, m2)
        r2 = _bfly(m1, jnp.minimum)
        e2 = m1 == r2
        m1 = jnp.where(e2, m2, m1)
        return _bfly(m1, jnp.minimum)

    def quad_body(jq, _):
        j0 = jq * 4
        g = j0 // _L
        sels = [lane == (j0 % _L + i) for i in range(4)]
        qxs = [_qcoord(qx, g, s) for s in sels]
        qys = [_qcoord(qy, g, s) for s in sels]
        qzs = [_qcoord(qz, g, s) for s in sels]
        dbufs = (db0, db1, db2, db3)

        def p1_body(c, ms):
            off = c * _L
            kxc = kx[pl.ds(off, _L)]
            kyc = ky[pl.ds(off, _L)]
            kzc = kz[pl.ds(off, _L)]
            res = []
            for i in range(4):
                m1, m2, m3 = ms[3 * i:3 * i + 3]
                dx = kxc - qxs[i]
                dy = kyc - qys[i]
                dz = kzc - qzs[i]
                d = (dx * dx + dy * dy) + dz * dz
                dbufs[i][pl.ds(off, _L)] = d
                res.extend(_ladder(m1, m2, m3, d))
            return tuple(res)

        ms = lax.fori_loop(0, _N // _L, p1_body, (inf16,) * 12)
        thrs = [_thresh(*ms[3 * i:3 * i + 3]) for i in range(4)]

        for ref in (a2r, a2g, a2b, a3r, a3g, a3b):
            ref[pl.ds(0, _L)] = zero16

        def p2_body(c, acc):
            ar0, ag0, ab0, cn0, ar1, ag1, ab1, cn1, cn2, cn3 = acc
            off = c * _L
            crc = cr[pl.ds(off, _L)]
            cgc = cg[pl.ds(off, _L)]
            cbc = cb[pl.ds(off, _L)]
            s0 = db0[pl.ds(off, _L)] <= thrs[0]
            ar0 = ar0 + jnp.where(s0, crc, zero16)
            ag0 = ag0 + jnp.where(s0, cgc, zero16)
            ab0 = ab0 + jnp.where(s0, cbc, zero16)
            cn0 = cn0 + plsc.all_reduce_population_count(s0)
            s1 = db1[pl.ds(off, _L)] <= thrs[1]
            ar1 = ar1 + jnp.where(s1, crc, zero16)
            ag1 = ag1 + jnp.where(s1, cgc, zero16)
            ab1 = ab1 + jnp.where(s1, cbc, zero16)
            cn1 = cn1 + plsc.all_reduce_population_count(s1)
            s2 = db2[pl.ds(off, _L)] <= thrs[2]
            plsc.addupdate(a2r.at[pl.ds(0, _L)], jnp.where(s2, crc, zero16))
            plsc.addupdate(a2g.at[pl.ds(0, _L)], jnp.where(s2, cgc, zero16))
            plsc.addupdate(a2b.at[pl.ds(0, _L)], jnp.where(s2, cbc, zero16))
            cn2 = cn2 + plsc.all_reduce_population_count(s2)
            s3 = db3[pl.ds(off, _L)] <= thrs[3]
            plsc.addupdate(a3r.at[pl.ds(0, _L)], jnp.where(s3, crc, zero16))
            plsc.addupdate(a3g.at[pl.ds(0, _L)], jnp.where(s3, cgc, zero16))
            plsc.addupdate(a3b.at[pl.ds(0, _L)], jnp.where(s3, cbc, zero16))
            cn3 = cn3 + plsc.all_reduce_population_count(s3)
            return (ar0, ag0, ab0, cn0, ar1, ag1, ab1, cn1, cn2, cn3)

        ar0, ag0, ab0, cn0, ar1, ag1, ab1, cn1, cn2, cn3 = lax.fori_loop(
            0, _N // _L, p2_body,
            (zero16, zero16, zero16, izero,
             zero16, zero16, zero16, izero, izero, izero))

        rs, gs, bs = [], [], []
        for i, (ar, ag, ab, cn) in enumerate((
                (ar0, ag0, ab0, cn0),
                (ar1, ag1, ab1, cn1),
                (a2r[pl.ds(0, _L)], a2g[pl.ds(0, _L)], a2b[pl.ds(0, _L)],
                 cn2),
                (a3r[pl.ds(0, _L)], a3g[pl.ds(0, _L)], a3b[pl.ds(0, _L)],
                 cn3))):
            inv = one16 / cn.astype(jnp.float32)
            rs.append(_bfly(ar, jnp.add) * inv)
            gs.append(_bfly(ag, jnp.add) * inv)
            bs.append(_bfly(ab, jnp.add) * inv)

        def _merge(vals, old):
            res = old
            for i in range(4):
                res = jnp.where(sels[i], vals[i], res)
            return res

        ovr[pl.ds(g * _L, _L)] = _merge(rs, ovr[pl.ds(g * _L, _L)])
        ovg[pl.ds(g * _L, _L)] = _merge(gs, ovg[pl.ds(g * _L, _L)])
        ovb[pl.ds(g * _L, _L)] = _merge(bs, ovb[pl.ds(g * _L, _L)])
        return 0

    lax.fori_loop(0, qpw // 4, quad_body, 0)

    base = b * _N + qoff
    pltpu.sync_copy(ovr, out.at[pl.ds(0 * _NQ + base, qpw)])
    pltpu.sync_copy(ovg, out.at[pl.ds(1 * _NQ + base, qpw)])
    pltpu.sync_copy(ovb, out.at[pl.ds(2 * _NQ + base, qpw)])


def kernel(points1, points2, colors1):
    f32 = jnp.float32
    p1t = jnp.transpose(points1, (0, 2, 1)).reshape(_B * 3 * _N)
    p2t = jnp.transpose(points2, (0, 2, 1)).reshape(_B * 3 * _N)
    c1t = jnp.transpose(colors1, (0, 2, 1)).reshape(_B * 3 * _N)

    mesh = plsc.VectorSubcoreMesh(core_axis_name="c", subcore_axis_name="s")
    sc = functools.partial(
        pl.kernel,
        mesh=mesh,
        compiler_params=pltpu.CompilerParams(needs_layout_passes=False),
        out_type=jax.ShapeDtypeStruct((3 * _NQ,), f32),
        scratch_types=[
            pltpu.VMEM((_N,), f32),
            pltpu.VMEM((_N,), f32),
            pltpu.VMEM((_N,), f32),
            pltpu.VMEM((_N,), f32),
            pltpu.VMEM((_N,), f32),
            pltpu.VMEM((_N,), f32),
            pltpu.VMEM((_NQ // 32,), f32),
            pltpu.VMEM((_NQ // 32,), f32),
            pltpu.VMEM((_NQ // 32,), f32),
            pltpu.VMEM((_N,), f32),
            pltpu.VMEM((_N,), f32),
            pltpu.VMEM((_N,), f32),
            pltpu.VMEM((_N,), f32),
            pltpu.VMEM((_L,), f32),
            pltpu.VMEM((_L,), f32),
            pltpu.VMEM((_L,), f32),
            pltpu.VMEM((_L,), f32),
            pltpu.VMEM((_L,), f32),
            pltpu.VMEM((_L,), f32),
            pltpu.VMEM((_NQ // 32,), f32),
            pltpu.VMEM((_NQ // 32,), f32),
            pltpu.VMEM((_NQ // 32,), f32),
        ],
    )(_sc_knn)
    out_t = sc(p1t, p2t, c1t)
    return jnp.transpose(out_t.reshape(3, _B, _N), (1, 2, 0))


# speedup vs baseline: 1.2278x; 1.2278x over previous
import functools

import jax
import jax.numpy as jnp
from jax import lax
from jax.experimental import pallas as pl
from jax.experimental.pallas import tpu as pltpu
from jax.experimental.pallas import tpu_sc as plsc

_B = 4
_N = 2048
_NQ = _B * _N
_L = 16


def _ladder(m1, m2, m3, d):
    t = jnp.maximum(m1, d)
    m1 = jnp.minimum(m1, d)
    t2 = jnp.maximum(m2, t)
    m2 = jnp.minimum(m2, t)
    m3 = jnp.minimum(m3, t2)
    return m1, m2, m3


def _sc_knn(p1t, p2t, c1t, out, kx, ky, kz, cr, cg, cb, qx, qy, qz,
            db0, db1, ovr, ovg, ovb):
    nw = 32
    qpw = _NQ // nw
    wid = lax.axis_index("c") * 16 + lax.axis_index("s")
    b = wid // (_N // qpw)
    qoff = (wid % (_N // qpw)) * qpw

    pltpu.sync_copy(p1t.at[pl.ds((b * 3 + 0) * _N, _N)], kx)
    pltpu.sync_copy(p1t.at[pl.ds((b * 3 + 1) * _N, _N)], ky)
    pltpu.sync_copy(p1t.at[pl.ds((b * 3 + 2) * _N, _N)], kz)
    pltpu.sync_copy(c1t.at[pl.ds((b * 3 + 0) * _N, _N)], cr)
    pltpu.sync_copy(c1t.at[pl.ds((b * 3 + 1) * _N, _N)], cg)
    pltpu.sync_copy(c1t.at[pl.ds((b * 3 + 2) * _N, _N)], cb)
    pltpu.sync_copy(p2t.at[pl.ds((b * 3 + 0) * _N + qoff, qpw)], qx)
    pltpu.sync_copy(p2t.at[pl.ds((b * 3 + 1) * _N + qoff, qpw)], qy)
    pltpu.sync_copy(p2t.at[pl.ds((b * 3 + 2) * _N + qoff, qpw)], qz)

    inf16 = jnp.full((_L,), jnp.inf, jnp.float32)
    zero16 = jnp.zeros((_L,), jnp.float32)
    one16 = jnp.ones((_L,), jnp.float32)

    lane = lax.iota(jnp.int32, _L)
    ninf16 = jnp.full((_L,), -jnp.inf, jnp.float32)

    _dn = lax.GatherDimensionNumbers(
        offset_dims=(), collapsed_slice_dims=(0,), start_index_map=(0,))

    def _shuf(v, idx):
        return lax.gather(v, idx[:, None], _dn, (1,),
                          mode=lax.GatherScatterMode.PROMISE_IN_BOUNDS)

    def _bfly(v, op):
        for s in (8, 4, 2, 1):
            v = op(v, _shuf(v, lane ^ s))
        return v

    def _qcoord(qv, g, sel):
        return _bfly(jnp.where(sel, qv[pl.ds(g * _L, _L)], ninf16),
                     jnp.maximum)

    def _thresh(m1, m2, m3):
        r1 = _bfly(m1, jnp.minimum)
        e1 = m1 == r1
        m1 = jnp.where(e1, m2, m1)
        m2 = jnp.where(e1, m3, m2)
        r2 = _bfly(m1, jnp.minimum)
        e2 = m1 == r2
        m1 = jnp.where(e2, m2, m1)
        return _bfly(m1, jnp.minimum)

    def pair_body(jp, _):
        j0 = jp * 2
        g = j0 // _L
        sel0 = lane == (j0 % _L)
        sel1 = lane == (j0 % _L + 1)
        qx0 = _qcoord(qx, g, sel0)
        qy0 = _qcoord(qy, g, sel0)
        qz0 = _qcoord(qz, g, sel0)
        qx1 = _qcoord(qx, g, sel1)
        qy1 = _qcoord(qy, g, sel1)
        qz1 = _qcoord(qz, g, sel1)

        def _dist(kxc, kyc, kzc, qxv, qyv, qzv):
            dx = kxc - qxv
            dy = kyc - qyv
            dz = kzc - qzv
            return (dx * dx + dy * dy) + dz * dz

        def p1_body(c, ms):
            m10, m20, m30, m11, m21, m31 = ms
            off = c * 32
            for half in (0, 1):
                o = off + half * _L
                kxc = kx[pl.ds(o, _L)]
                kyc = ky[pl.ds(o, _L)]
                kzc = kz[pl.ds(o, _L)]
                da = _dist(kxc, kyc, kzc, qx0, qy0, qz0)
                db0[pl.ds(o, _L)] = da
                dbv = _dist(kxc, kyc, kzc, qx1, qy1, qz1)
                db1[pl.ds(o, _L)] = dbv
                m10, m20, m30 = _ladder(m10, m20, m30, da)
                m11, m21, m31 = _ladder(m11, m21, m31, dbv)
            return (m10, m20, m30, m11, m21, m31)

        m10, m20, m30, m11, m21, m31 = lax.fori_loop(
            0, _N // 32, p1_body, (inf16,) * 6)
        thr0 = _thresh(m10, m20, m30)
        thr1 = _thresh(m11, m21, m31)

        def p2_body(c, acc):
            ar0, ag0, ab0, cn0, ar1, ag1, ab1, cn1 = acc
            off = c * 32
            for half in (0, 1):
                o = off + half * _L
                crc = cr[pl.ds(o, _L)]
                cgc = cg[pl.ds(o, _L)]
                cbc = cb[pl.ds(o, _L)]
                s0 = db0[pl.ds(o, _L)] <= thr0
                ar0 = ar0 + jnp.where(s0, crc, zero16)
                ag0 = ag0 + jnp.where(s0, cgc, zero16)
                ab0 = ab0 + jnp.where(s0, cbc, zero16)
                cn0 = cn0 + plsc.all_reduce_population_count(s0)
                s1 = db1[pl.ds(o, _L)] <= thr1
                ar1 = ar1 + jnp.where(s1, crc, zero16)
                ag1 = ag1 + jnp.where(s1, cgc, zero16)
                ab1 = ab1 + jnp.where(s1, cbc, zero16)
                cn1 = cn1 + plsc.all_reduce_population_count(s1)
            return (ar0, ag0, ab0, cn0, ar1, ag1, ab1, cn1)

        izero = jnp.zeros((_L,), jnp.int32)
        ar0, ag0, ab0, cn0, ar1, ag1, ab1, cn1 = lax.fori_loop(
            0, _N // 32, p2_body,
            (zero16, zero16, zero16, izero, zero16, zero16, zero16, izero))

        inv0 = one16 / cn0.astype(jnp.float32)
        inv1 = one16 / cn1.astype(jnp.float32)
        r0 = _bfly(ar0, jnp.add) * inv0
        g0 = _bfly(ag0, jnp.add) * inv0
        b0 = _bfly(ab0, jnp.add) * inv0
        r1 = _bfly(ar1, jnp.add) * inv1
        g1 = _bfly(ag1, jnp.add) * inv1
        b1 = _bfly(ab1, jnp.add) * inv1
        ovr[pl.ds(g * _L, _L)] = jnp.where(
            sel0, r0, jnp.where(sel1, r1, ovr[pl.ds(g * _L, _L)]))
        ovg[pl.ds(g * _L, _L)] = jnp.where(
            sel0, g0, jnp.where(sel1, g1, ovg[pl.ds(g * _L, _L)]))
        ovb[pl.ds(g * _L, _L)] = jnp.where(
            sel0, b0, jnp.where(sel1, b1, ovb[pl.ds(g * _L, _L)]))
        return 0

    lax.fori_loop(0, qpw // 2, pair_body, 0)

    base = b * _N + qoff
    pltpu.sync_copy(ovr, out.at[pl.ds(0 * _NQ + base, qpw)])
    pltpu.sync_copy(ovg, out.at[pl.ds(1 * _NQ + base, qpw)])
    pltpu.sync_copy(ovb, out.at[pl.ds(2 * _NQ + base, qpw)])


def kernel(points1, points2, colors1):
    f32 = jnp.float32
    p1t = jnp.transpose(points1, (0, 2, 1)).reshape(_B * 3 * _N)
    p2t = jnp.transpose(points2, (0, 2, 1)).reshape(_B * 3 * _N)
    c1t = jnp.transpose(colors1, (0, 2, 1)).reshape(_B * 3 * _N)

    mesh = plsc.VectorSubcoreMesh(core_axis_name="c", subcore_axis_name="s")
    sc = functools.partial(
        pl.kernel,
        mesh=mesh,
        compiler_params=pltpu.CompilerParams(needs_layout_passes=False),
        out_type=jax.ShapeDtypeStruct((3 * _NQ,), f32),
        scratch_types=[
            pltpu.VMEM((_N,), f32),
            pltpu.VMEM((_N,), f32),
            pltpu.VMEM((_N,), f32),
            pltpu.VMEM((_N,), f32),
            pltpu.VMEM((_N,), f32),
            pltpu.VMEM((_N,), f32),
            pltpu.VMEM((_NQ // 32,), f32),
            pltpu.VMEM((_NQ // 32,), f32),
            pltpu.VMEM((_NQ // 32,), f32),
            pltpu.VMEM((_N,), f32),
            pltpu.VMEM((_N,), f32),
            pltpu.VMEM((_NQ // 32,), f32),
            pltpu.VMEM((_NQ // 32,), f32),
            pltpu.VMEM((_NQ // 32,), f32),
        ],
    )(_sc_knn)
    out_t = sc(p1t, p2t, c1t)
    return jnp.transpose(out_t.reshape(3, _B, _N), (1, 2, 0))


# speedup vs baseline: 1.2863x; 1.0476x over previous
import functools

import jax
import jax.numpy as jnp
from jax import lax
from jax.experimental import pallas as pl
from jax.experimental.pallas import tpu as pltpu
from jax.experimental.pallas import tpu_sc as plsc

_B = 4
_N = 2048
_NQ = _B * _N
_L = 16


def _ladder(m1, m2, m3, d):
    t = jnp.maximum(m1, d)
    m1 = jnp.minimum(m1, d)
    t2 = jnp.maximum(m2, t)
    m2 = jnp.minimum(m2, t)
    m3 = jnp.minimum(m3, t2)
    return m1, m2, m3


def _ladder2(m1, m2, d):
    t = jnp.maximum(m1, d)
    m1 = jnp.minimum(m1, d)
    m2 = jnp.minimum(m2, t)
    return m1, m2


def _sc_knn(p1t, p2t, c1t, out, kx, ky, kz, cr, cg, cb, qx, qy, qz,
            db0, db1, ovr, ovg, ovb):
    nw = 32
    qpw = _NQ // nw
    wid = lax.axis_index("c") * 16 + lax.axis_index("s")
    b = wid // (_N // qpw)
    qoff = (wid % (_N // qpw)) * qpw

    pltpu.sync_copy(p1t.at[pl.ds((b * 3 + 0) * _N, _N)], kx)
    pltpu.sync_copy(p1t.at[pl.ds((b * 3 + 1) * _N, _N)], ky)
    pltpu.sync_copy(p1t.at[pl.ds((b * 3 + 2) * _N, _N)], kz)
    pltpu.sync_copy(c1t.at[pl.ds((b * 3 + 0) * _N, _N)], cr)
    pltpu.sync_copy(c1t.at[pl.ds((b * 3 + 1) * _N, _N)], cg)
    pltpu.sync_copy(c1t.at[pl.ds((b * 3 + 2) * _N, _N)], cb)
    pltpu.sync_copy(p2t.at[pl.ds((b * 3 + 0) * _N + qoff, qpw)], qx)
    pltpu.sync_copy(p2t.at[pl.ds((b * 3 + 1) * _N + qoff, qpw)], qy)
    pltpu.sync_copy(p2t.at[pl.ds((b * 3 + 2) * _N + qoff, qpw)], qz)

    inf16 = jnp.full((_L,), jnp.inf, jnp.float32)
    zero16 = jnp.zeros((_L,), jnp.float32)
    one16 = jnp.ones((_L,), jnp.float32)

    lane = lax.iota(jnp.int32, _L)
    ninf16 = jnp.full((_L,), -jnp.inf, jnp.float32)

    _dn = lax.GatherDimensionNumbers(
        offset_dims=(), collapsed_slice_dims=(0,), start_index_map=(0,))

    def _shuf(v, idx):
        return lax.gather(v, idx[:, None], _dn, (1,),
                          mode=lax.GatherScatterMode.PROMISE_IN_BOUNDS)

    def _bfly(v, op):
        for s in (8, 4, 2, 1):
            v = op(v, _shuf(v, lane ^ s))
        return v

    def _qcoord(qv, g, sel):
        return _bfly(jnp.where(sel, qv[pl.ds(g * _L, _L)], ninf16),
                     jnp.maximum)

    def _thresh2(m1, m2):
        r1 = _bfly(m1, jnp.minimum)
        e1 = m1 == r1
        m1 = jnp.where(e1, m2, m1)
        m2 = jnp.where(e1, inf16, m2)
        r2 = _bfly(m1, jnp.minimum)
        e2 = m1 == r2
        m1 = jnp.where(e2, m2, m1)
        return _bfly(m1, jnp.minimum)

    def _thresh(m1, m2, m3):
        r1 = _bfly(m1, jnp.minimum)
        e1 = m1 == r1
        m1 = jnp.where(e1, m2, m1)
        m2 = jnp.where(e1, m3, m2)
        r2 = _bfly(m1, jnp.minimum)
        e2 = m1 == r2
        m1 = jnp.where(e2, m2, m1)
        return _bfly(m1, jnp.minimum)

    def pair_body(jp, _):
        j0 = jp * 2
        g = j0 // _L
        sel0 = lane == (j0 % _L)
        sel1 = lane == (j0 % _L + 1)
        qx0 = _qcoord(qx, g, sel0)
        qy0 = _qcoord(qy, g, sel0)
        qz0 = _qcoord(qz, g, sel0)
        qx1 = _qcoord(qx, g, sel1)
        qy1 = _qcoord(qy, g, sel1)
        qz1 = _qcoord(qz, g, sel1)

        def _dist(kxc, kyc, kzc, qxv, qyv, qzv):
            dx = kxc - qxv
            dy = kyc - qyv
            dz = kzc - qzv
            return (dx * dx + dy * dy) + dz * dz

        def p1_body(c, ms):
            m10, m20, m11, m21 = ms
            off = c * 32
            for half in (0, 1):
                o = off + half * _L
                kxc = kx[pl.ds(o, _L)]
                kyc = ky[pl.ds(o, _L)]
                kzc = kz[pl.ds(o, _L)]
                da = _dist(kxc, kyc, kzc, qx0, qy0, qz0)
                db0[pl.ds(o, _L)] = da
                dbv = _dist(kxc, kyc, kzc, qx1, qy1, qz1)
                db1[pl.ds(o, _L)] = dbv
                m10, m20 = _ladder2(m10, m20, da)
                m11, m21 = _ladder2(m11, m21, dbv)
            return (m10, m20, m11, m21)

        m10, m20, m11, m21 = lax.fori_loop(
            0, _N // 32, p1_body, (inf16,) * 4)
        thr0 = _thresh2(m10, m20)
        thr1 = _thresh2(m11, m21)

        def p2_body(c, acc):
            ar0, ag0, ab0, cn0, ar1, ag1, ab1, cn1 = acc
            off = c * 32
            for half in (0, 1):
                o = off + half * _L
                crc = cr[pl.ds(o, _L)]
                cgc = cg[pl.ds(o, _L)]
                cbc = cb[pl.ds(o, _L)]
                s0 = db0[pl.ds(o, _L)] <= thr0
                ar0 = ar0 + jnp.where(s0, crc, zero16)
                ag0 = ag0 + jnp.where(s0, cgc, zero16)
                ab0 = ab0 + jnp.where(s0, cbc, zero16)
                cn0 = cn0 + plsc.all_reduce_population_count(s0)
                s1 = db1[pl.ds(o, _L)] <= thr1
                ar1 = ar1 + jnp.where(s1, crc, zero16)
                ag1 = ag1 + jnp.where(s1, cgc, zero16)
                ab1 = ab1 + jnp.where(s1, cbc, zero16)
                cn1 = cn1 + plsc.all_reduce_population_count(s1)
            return (ar0, ag0, ab0, cn0, ar1, ag1, ab1, cn1)

        izero = jnp.zeros((_L,), jnp.int32)
        ar0, ag0, ab0, cn0, ar1, ag1, ab1, cn1 = lax.fori_loop(
            0, _N // 32, p2_body,
            (zero16, zero16, zero16, izero, zero16, zero16, zero16, izero))

        def _exact_one(dbuf):
            def lad_body(c, ms):
                m1, m2, m3 = ms
                m1, m2, m3 = _ladder(m1, m2, m3, dbuf[pl.ds(c * 32, _L)])
                return _ladder(m1, m2, m3, dbuf[pl.ds(c * 32 + _L, _L)])

            m1, m2, m3 = lax.fori_loop(0, _N // 32, lad_body, (inf16,) * 3)
            thr = _thresh(m1, m2, m3)

            def acc_body(c, acc):
                ar, ag, ab, cn = acc
                for half in (0, 1):
                    o = c * 32 + half * _L
                    s = dbuf[pl.ds(o, _L)] <= thr
                    ar = ar + jnp.where(s, cr[pl.ds(o, _L)], zero16)
                    ag = ag + jnp.where(s, cg[pl.ds(o, _L)], zero16)
                    ab = ab + jnp.where(s, cb[pl.ds(o, _L)], zero16)
                    cn = cn + plsc.all_reduce_population_count(s)
                return (ar, ag, ab, cn)

            return lax.fori_loop(
                0, _N // 32, acc_body,
                (zero16, zero16, zero16, jnp.zeros((_L,), jnp.int32)))

        ar0, ag0, ab0, cn0 = lax.cond(
            jnp.max(cn0) == 3,
            lambda: (ar0, ag0, ab0, cn0),
            lambda: _exact_one(db0))
        ar1, ag1, ab1, cn1 = lax.cond(
            jnp.max(cn1) == 3,
            lambda: (ar1, ag1, ab1, cn1),
            lambda: _exact_one(db1))

        inv0 = one16 / cn0.astype(jnp.float32)
        inv1 = one16 / cn1.astype(jnp.float32)
        r0 = _bfly(ar0, jnp.add) * inv0
        g0 = _bfly(ag0, jnp.add) * inv0
        b0 = _bfly(ab0, jnp.add) * inv0
        r1 = _bfly(ar1, jnp.add) * inv1
        g1 = _bfly(ag1, jnp.add) * inv1
        b1 = _bfly(ab1, jnp.add) * inv1
        ovr[pl.ds(g * _L, _L)] = jnp.where(
            sel0, r0, jnp.where(sel1, r1, ovr[pl.ds(g * _L, _L)]))
        ovg[pl.ds(g * _L, _L)] = jnp.where(
            sel0, g0, jnp.where(sel1, g1, ovg[pl.ds(g * _L, _L)]))
        ovb[pl.ds(g * _L, _L)] = jnp.where(
            sel0, b0, jnp.where(sel1, b1, ovb[pl.ds(g * _L, _L)]))
        return 0

    lax.fori_loop(0, qpw // 2, pair_body, 0)

    base = b * _N + qoff
    pltpu.sync_copy(ovr, out.at[pl.ds(0 * _NQ + base, qpw)])
    pltpu.sync_copy(ovg, out.at[pl.ds(1 * _NQ + base, qpw)])
    pltpu.sync_copy(ovb, out.at[pl.ds(2 * _NQ + base, qpw)])


def kernel(points1, points2, colors1):
    f32 = jnp.float32
    p1t = jnp.transpose(points1, (0, 2, 1)).reshape(_B * 3 * _N)
    p2t = jnp.transpose(points2, (0, 2, 1)).reshape(_B * 3 * _N)
    c1t = jnp.transpose(colors1, (0, 2, 1)).reshape(_B * 3 * _N)

    mesh = plsc.VectorSubcoreMesh(core_axis_name="c", subcore_axis_name="s")
    sc = functools.partial(
        pl.kernel,
        mesh=mesh,
        compiler_params=pltpu.CompilerParams(needs_layout_passes=False),
        out_type=jax.ShapeDtypeStruct((3 * _NQ,), f32),
        scratch_types=[
            pltpu.VMEM((_N,), f32),
            pltpu.VMEM((_N,), f32),
            pltpu.VMEM((_N,), f32),
            pltpu.VMEM((_N,), f32),
            pltpu.VMEM((_N,), f32),
            pltpu.VMEM((_N,), f32),
            pltpu.VMEM((_NQ // 32,), f32),
            pltpu.VMEM((_NQ // 32,), f32),
            pltpu.VMEM((_NQ // 32,), f32),
            pltpu.VMEM((_N,), f32),
            pltpu.VMEM((_N,), f32),
            pltpu.VMEM((_NQ // 32,), f32),
            pltpu.VMEM((_NQ // 32,), f32),
            pltpu.VMEM((_NQ // 32,), f32),
        ],
    )(_sc_knn)
    out_t = sc(p1t, p2t, c1t)
    return jnp.transpose(out_t.reshape(3, _B, _N), (1, 2, 0))
